# Initial kernel scaffold; baseline (speedup 1.0000x reference)
import jax
import jax.numpy as jnp
from jax.experimental import pallas as pl


def kernel(queries, keys, k):
    raise NotImplementedError("write your pallas kernel here")


# speedup vs baseline: 2.7382x; 2.7382x over previous
import jax
import jax.numpy as jnp
from jax.experimental import pallas as pl
from jax.experimental.pallas import tpu as pltpu


_BK = 512


def _nn_body(q_ref, kt_ref, dist_ref, idx_ref, m_ref, mi_ref):
    i = pl.program_id(0)
    nblk = pl.num_programs(0)

    qx = q_ref[:, 0:1]
    qy = q_ref[:, 1:2]
    qz = q_ref[:, 2:3]
    x2 = (qx * qx + qy * qy) + qz * qz

    kx = kt_ref[0:1, :]
    ky = kt_ref[1:2, :]
    kz = kt_ref[2:3, :]
    y2 = (kx * kx + ky * ky) + kz * kz

    qb = q_ref[...].astype(jnp.bfloat16)
    kb = kt_ref[...].astype(jnp.bfloat16)
    t = jax.lax.dot_general(
        qb, kb, (((1,), (0,)), ((), ())),
        preferred_element_type=jnp.float32)
    d2 = (x2 + y2) - 2.0 * t
    d = jnp.sqrt(jnp.maximum(d2, 1e-12))

    kidx = jax.lax.broadcasted_iota(jnp.int32, (1, _BK), 1) + i * _BK

    @pl.when(i == 0)
    def _init():
        m_ref[...] = d
        mi_ref[...] = jnp.broadcast_to(kidx, d.shape)

    @pl.when(i > 0)
    def _update():
        m = m_ref[...]
        mask = d < m
        m_ref[...] = jnp.where(mask, d, m)
        mi_ref[...] = jnp.where(mask, kidx, mi_ref[...])

    @pl.when(i == nblk - 1)
    def _finalize():
        m = m_ref[...]
        v = jnp.min(m, axis=1, keepdims=True)
        cand = jnp.where(m == v, mi_ref[...], jnp.int32(2**31 - 1))
        dist_ref[...] = v
        idx_ref[...] = jnp.min(cand, axis=1, keepdims=True)


def kernel(queries, keys, k):
    q, dim = queries.shape
    nk = keys.shape[0]
    pdim = 8
    queries_p = jnp.pad(queries, ((0, 0), (0, pdim - dim)))
    keys_t = jnp.pad(keys.T, ((0, pdim - dim), (0, 0)))
    nblk = nk // _BK

    dist, idx = pl.pallas_call(
        _nn_body,
        grid=(nblk,),
        in_specs=[
            pl.BlockSpec((q, pdim), lambda i: (0, 0)),
            pl.BlockSpec((pdim, _BK), lambda i: (0, i)),
        ],
        out_specs=[
            pl.BlockSpec((q, 1), lambda i: (0, 0)),
            pl.BlockSpec((q, 1), lambda i: (0, 0)),
        ],
        out_shape=[
            jax.ShapeDtypeStruct((q, 1), jnp.float32),
            jax.ShapeDtypeStruct((q, 1), jnp.int32),
        ],
        scratch_shapes=[
            pltpu.VMEM((q, _BK), jnp.float32),
            pltpu.VMEM((q, _BK), jnp.int32),
        ],
    )(queries_p, keys_t)

    idx = idx + (jnp.asarray(k, dtype=idx.dtype) - 1)
    return (dist, idx, idx[:, 0])
